# Initial kernel scaffold; baseline (speedup 1.0000x reference)
import jax
import jax.numpy as jnp
from jax.experimental import pallas as pl


def kernel(feat, edge_index, W_neigh, W_self, b_self):
    raise NotImplementedError("write your pallas kernel here")


# speedup vs baseline: 3.4956x; 3.4956x over previous
import functools

import jax
import jax.numpy as jnp
from jax import lax
from jax.experimental import pallas as pl
from jax.experimental.pallas import tpu as pltpu
from jax.experimental.pallas import tpu_sc as plsc

N = 10000
E = 160000
D = 256
DH = 128
NS = 16
NPAD = 10240
RPW = NPAD // NS
EPS = E // NS
C = 80
NITER = EPS // C
NCHUNKS = E // C
P2ITERS = -(-NCHUNKS // 32)


def _sc_segment_sum(featL, featR, src, dst, zacc, ones128):
    mesh = plsc.VectorSubcoreMesh(core_axis_name="c", subcore_axis_name="s")
    out_type = (
        jax.ShapeDtypeStruct((NPAD, DH), jnp.float32),
        jax.ShapeDtypeStruct((NPAD, DH), jnp.float32),
        jax.ShapeDtypeStruct((NPAD, DH), jnp.float32),
        jax.ShapeDtypeStruct((NPAD, DH), jnp.float32),
    )

    @functools.partial(
        pl.kernel, mesh=mesh, out_type=out_type,
        scratch_types=[
            pltpu.VMEM_SHARED((NPAD, DH), jnp.float32),
            pltpu.VMEM((C,), jnp.int32),
            pltpu.VMEM((C,), jnp.int32),
            pltpu.VMEM((C, DH), jnp.float32),
            pltpu.VMEM((C, DH), jnp.float32),
            pltpu.SemaphoreType.DMA,
        ],
    )
    def k(featL_hbm, featR_hbm, src_hbm, dst_hbm, zacc_hbm, ones_hbm,
          sumL_hbm, sumR_hbm, degA_hbm, degB_hbm,
          acc_sh, src_v, dst_v, rows_v, ones_v, gsem):
        cid = lax.axis_index("c")
        sid = lax.axis_index("s")
        wid = cid * NS + sid
        r0 = sid * RPW
        nchunk = RPW // C

        def zero_acc():
            pltpu.sync_copy(zacc_hbm, rows_v)

            @pl.loop(0, nchunk)
            def _(j):
                pltpu.sync_copy(rows_v, acc_sh.at[pl.ds(r0 + j * C, C)])

        def write_acc(out_hbm):
            @pl.loop(0, nchunk)
            def _(j):
                rr = r0 + j * C
                pltpu.sync_copy(acc_sh.at[pl.ds(rr, C)], rows_v)
                pltpu.sync_copy(rows_v, out_hbm.at[pl.ds(rr, C)])

        zero_acc()
        pltpu.sync_copy(ones_hbm, ones_v)
        plsc.subcore_barrier()

        def run(feat_hbm):
            @pl.loop(0, NITER)
            def _(i):
                base = sid * EPS + i * C
                pltpu.sync_copy(src_hbm.at[pl.ds(base, C)], src_v)
                pltpu.sync_copy(dst_hbm.at[pl.ds(base, C)], dst_v)
                pltpu.async_copy(feat_hbm.at[src_v], rows_v, gsem).wait()
                pltpu.sync_copy(rows_v, acc_sh.at[dst_v], add=True)

        @pl.when(cid == 0)
        def _():
            run(featL_hbm)

        @pl.when(cid == 1)
        def _():
            run(featR_hbm)

        plsc.subcore_barrier()

        @pl.when(cid == 0)
        def _():
            write_acc(sumL_hbm)

        @pl.when(cid == 1)
        def _():
            write_acc(sumR_hbm)

        plsc.subcore_barrier()

        zero_acc()
        plsc.subcore_barrier()

        @pl.loop(0, P2ITERS)
        def _(i):
            k_id = wid + 32 * i

            @pl.when(k_id < NCHUNKS)
            def _():
                pltpu.sync_copy(dst_hbm.at[pl.ds(k_id * C, C)], dst_v)
                pltpu.sync_copy(ones_v, acc_sh.at[dst_v], add=True)

        plsc.subcore_barrier()

        @pl.when(cid == 0)
        def _():
            write_acc(degA_hbm)

        @pl.when(cid == 1)
        def _():
            write_acc(degB_hbm)

    return k(featL, featR, src, dst, zacc, ones128)


def _tc_combine(feat, sumL, sumR, degA, degB, W_neigh, W_self, b_self):
    B = 400
    dn = (((1,), (1,)), ((), ()))

    def body(feat_ref, sL_ref, sR_ref, dA_ref, dB_ref, wn_ref, ws_ref, b_ref,
             o_ref):
        ns = jnp.concatenate([sL_ref[...], sR_ref[...]], axis=1)
        deg = jnp.maximum(dA_ref[...][:, 0:1] + dB_ref[...][:, 0:1], 1.0)
        h_neigh = ns / deg
        o_ref[...] = (
            lax.dot_general(feat_ref[...], ws_ref[...], dn,
                            preferred_element_type=jnp.float32)
            + b_ref[...]
            + lax.dot_general(h_neigh, wn_ref[...], dn,
                              preferred_element_type=jnp.float32)
        )

    return pl.pallas_call(
        body,
        grid=(N // B,),
        in_specs=[
            pl.BlockSpec((B, D), lambda i: (i, 0)),
            pl.BlockSpec((B, DH), lambda i: (i, 0)),
            pl.BlockSpec((B, DH), lambda i: (i, 0)),
            pl.BlockSpec((B, DH), lambda i: (i, 0)),
            pl.BlockSpec((B, DH), lambda i: (i, 0)),
            pl.BlockSpec((D, D), lambda i: (0, 0)),
            pl.BlockSpec((D, D), lambda i: (0, 0)),
            pl.BlockSpec((1, D), lambda i: (0, 0)),
        ],
        out_specs=pl.BlockSpec((B, D), lambda i: (i, 0)),
        out_shape=jax.ShapeDtypeStruct((N, D), jnp.float32),
    )(feat, sumL, sumR, degA, degB, W_neigh, W_self, b_self.reshape(1, D))


def kernel(feat, edge_index, W_neigh, W_self, b_self):
    src = edge_index[0]
    dst = edge_index[1]
    featL = feat[:, :DH]
    featR = feat[:, DH:]
    zacc = jnp.zeros((C, DH), jnp.float32)
    ones128 = jnp.ones((C, DH), jnp.float32)
    sumL, sumR, degA, degB = _sc_segment_sum(featL, featR, src, dst,
                                             zacc, ones128)
    return _tc_combine(feat, sumL, sumR, degA, degB, W_neigh, W_self, b_self)


# speedup vs baseline: 5.3878x; 1.5413x over previous
import functools

import jax
import jax.numpy as jnp
from jax import lax
from jax.experimental import pallas as pl
from jax.experimental.pallas import tpu as pltpu
from jax.experimental.pallas import tpu_sc as plsc

N = 10000
E = 160000
D = 256
DH = 128
NS = 16
NPAD = 10240
RPW = NPAD // NS
EPS = E // NS
C = 80
NITER = EPS // C
NCHUNKS = E // C
P2ITERS = -(-NCHUNKS // 32)


def _sc_segment_sum(featL, featR, src, dst, zacc, ones128):
    mesh = plsc.VectorSubcoreMesh(core_axis_name="c", subcore_axis_name="s")
    out_type = (
        jax.ShapeDtypeStruct((NPAD, DH), jnp.float32),
        jax.ShapeDtypeStruct((NPAD, DH), jnp.float32),
        jax.ShapeDtypeStruct((NPAD, DH), jnp.float32),
        jax.ShapeDtypeStruct((NPAD, DH), jnp.float32),
    )

    @functools.partial(
        pl.kernel, mesh=mesh, out_type=out_type,
        scratch_types=[
            pltpu.VMEM_SHARED((NPAD, DH), jnp.float32),
            pltpu.VMEM((C,), jnp.int32),
            pltpu.VMEM((C,), jnp.int32),
            pltpu.VMEM((C,), jnp.int32),
            pltpu.VMEM((C,), jnp.int32),
            pltpu.VMEM((C, DH), jnp.float32),
            pltpu.VMEM((C, DH), jnp.float32),
            pltpu.SemaphoreType.DMA,
            pltpu.SemaphoreType.DMA,
            pltpu.SemaphoreType.DMA,
            pltpu.SemaphoreType.DMA,
            pltpu.SemaphoreType.DMA,
            pltpu.SemaphoreType.DMA,
            pltpu.SemaphoreType.DMA,
            pltpu.SemaphoreType.DMA,
        ],
    )
    def k(featL_hbm, featR_hbm, src_hbm, dst_hbm, zacc_hbm, ones_hbm,
          sumL_hbm, sumR_hbm, degA_hbm, degB_hbm,
          acc_sh, src_v0, src_v1, dst_v0, dst_v1, rows_v0, rows_v1,
          isem0, isem1, dsem0, dsem1, gsem0, gsem1, ssem0, ssem1):
        cid = lax.axis_index("c")
        sid = lax.axis_index("s")
        wid = cid * NS + sid
        r0 = sid * RPW
        nchunk = RPW // C
        sv = (src_v0, src_v1)
        dv = (dst_v0, dst_v1)
        rv = (rows_v0, rows_v1)
        isem = (isem0, isem1)
        dsem = (dsem0, dsem1)
        gsem = (gsem0, gsem1)
        ssem = (ssem0, ssem1)
        rows_v = rows_v0
        ones_v = rows_v1

        def zero_acc():
            pltpu.sync_copy(zacc_hbm, rows_v)

            @pl.loop(0, nchunk)
            def _(j):
                pltpu.sync_copy(rows_v, acc_sh.at[pl.ds(r0 + j * C, C)])

        def write_acc(out_hbm):
            @pl.loop(0, nchunk)
            def _(j):
                rr = r0 + j * C
                pltpu.sync_copy(acc_sh.at[pl.ds(rr, C)], rows_v)
                pltpu.sync_copy(rows_v, out_hbm.at[pl.ds(rr, C)])

        zero_acc()
        plsc.subcore_barrier()

        def run(feat_hbm):
            npair = (NITER - 1) // 2

            def isrc_start(i, b):
                pltpu.async_copy(
                    src_hbm.at[pl.ds(sid * EPS + i * C, C)], sv[b], isem[b])

            def isrc_wait(b):
                pltpu.make_async_copy(
                    src_hbm.at[pl.ds(0, C)], sv[b], isem[b]).wait()

            def dst_start(i, b):
                pltpu.async_copy(
                    dst_hbm.at[pl.ds(sid * EPS + i * C, C)], dv[b], dsem[b])

            def dst_wait(b):
                pltpu.make_async_copy(
                    dst_hbm.at[pl.ds(0, C)], dv[b], dsem[b]).wait()

            def gather_start(b):
                pltpu.async_copy(feat_hbm.at[sv[b]], rv[b], gsem[b])

            def gather_wait(b):
                pltpu.make_async_copy(feat_hbm.at[sv[b]], rv[b],
                                      gsem[b]).wait()

            def scat_start(b):
                pltpu.async_copy(rv[b], acc_sh.at[dv[b]], ssem[b], add=True)

            def scat_wait(b):
                pltpu.make_async_copy(rv[b], acc_sh.at[dv[b]],
                                      ssem[b]).wait()

            isrc_start(0, 0)
            dst_start(0, 0)
            isrc_wait(0)
            gather_start(0)
            isrc_start(1, 1)
            dst_start(1, 1)
            gather_wait(0)
            dst_wait(0)
            scat_start(0)
            isrc_wait(1)
            gather_start(1)
            isrc_start(2, 0)

            @pl.loop(0, npair)
            def _(p):
                i = 1 + 2 * p
                gather_wait(1)
                dst_wait(1)
                scat_start(1)
                scat_wait(0)
                dst_start(i + 1, 0)
                isrc_wait(0)
                gather_start(0)

                @pl.when(p < npair - 1)
                def _():
                    isrc_start(i + 2, 1)

                gather_wait(0)
                dst_wait(0)
                scat_start(0)
                scat_wait(1)

                @pl.when(p < npair - 1)
                def _():
                    dst_start(i + 2, 1)
                    isrc_wait(1)
                    gather_start(1)
                    isrc_start(i + 3, 0)

            scat_wait(0)

        @pl.when(cid == 0)
        def _():
            run(featL_hbm)

        @pl.when(cid == 1)
        def _():
            run(featR_hbm)

        plsc.subcore_barrier()

        @pl.when(cid == 0)
        def _():
            write_acc(sumL_hbm)

        @pl.when(cid == 1)
        def _():
            write_acc(sumR_hbm)

        plsc.subcore_barrier()

        zero_acc()
        pltpu.sync_copy(ones_hbm, ones_v)
        plsc.subcore_barrier()

        @pl.loop(0, P2ITERS)
        def _(i):
            k_id = wid + 32 * i

            @pl.when(k_id < NCHUNKS)
            def _():
                pltpu.sync_copy(dst_hbm.at[pl.ds(k_id * C, C)], dst_v0)
                pltpu.sync_copy(ones_v, acc_sh.at[dst_v0], add=True)

        plsc.subcore_barrier()

        @pl.when(cid == 0)
        def _():
            write_acc(degA_hbm)

        @pl.when(cid == 1)
        def _():
            write_acc(degB_hbm)

    return k(featL, featR, src, dst, zacc, ones128)


def _tc_combine(feat, sumL, sumR, degA, degB, W_neigh, W_self, b_self):
    B = 400
    dn = (((1,), (1,)), ((), ()))

    def body(feat_ref, sL_ref, sR_ref, dA_ref, dB_ref, wn_ref, ws_ref, b_ref,
             o_ref):
        ns = jnp.concatenate([sL_ref[...], sR_ref[...]], axis=1)
        deg = jnp.maximum(dA_ref[...][:, 0:1] + dB_ref[...][:, 0:1], 1.0)
        h_neigh = ns / deg
        o_ref[...] = (
            lax.dot_general(feat_ref[...], ws_ref[...], dn,
                            preferred_element_type=jnp.float32)
            + b_ref[...]
            + lax.dot_general(h_neigh, wn_ref[...], dn,
                              preferred_element_type=jnp.float32)
        )

    return pl.pallas_call(
        body,
        grid=(N // B,),
        in_specs=[
            pl.BlockSpec((B, D), lambda i: (i, 0)),
            pl.BlockSpec((B, DH), lambda i: (i, 0)),
            pl.BlockSpec((B, DH), lambda i: (i, 0)),
            pl.BlockSpec((B, DH), lambda i: (i, 0)),
            pl.BlockSpec((B, DH), lambda i: (i, 0)),
            pl.BlockSpec((D, D), lambda i: (0, 0)),
            pl.BlockSpec((D, D), lambda i: (0, 0)),
            pl.BlockSpec((1, D), lambda i: (0, 0)),
        ],
        out_specs=pl.BlockSpec((B, D), lambda i: (i, 0)),
        out_shape=jax.ShapeDtypeStruct((N, D), jnp.float32),
    )(feat, sumL, sumR, degA, degB, W_neigh, W_self, b_self.reshape(1, D))


def kernel(feat, edge_index, W_neigh, W_self, b_self):
    src = edge_index[0]
    dst = edge_index[1]
    featL = feat[:, :DH]
    featR = feat[:, DH:]
    zacc = jnp.zeros((C, DH), jnp.float32)
    ones128 = jnp.ones((C, DH), jnp.float32)
    sumL, sumR, degA, degB = _sc_segment_sum(featL, featR, src, dst,
                                             zacc, ones128)
    return _tc_combine(feat, sumL, sumR, degA, degB, W_neigh, W_self, b_self)


# speedup vs baseline: 5.9072x; 1.0964x over previous
import functools

import jax
import jax.numpy as jnp
from jax import lax
from jax.experimental import pallas as pl
from jax.experimental.pallas import tpu as pltpu
from jax.experimental.pallas import tpu_sc as plsc

N = 10000
E = 160000
D = 256
DH = 128
NS = 16
NPAD = 10240
RPW = NPAD // NS
EPS = E // NS
C = 80
NITER = EPS // C
NCHUNKS = E // C
P2ITERS = -(-NCHUNKS // 32)


def _sc_segment_sum(featL, featR, src, dst, zacc, ones128):
    mesh = plsc.VectorSubcoreMesh(core_axis_name="c", subcore_axis_name="s")
    out_type = (
        jax.ShapeDtypeStruct((NPAD, DH), jnp.float32),
        jax.ShapeDtypeStruct((NPAD, DH), jnp.float32),
        jax.ShapeDtypeStruct((NPAD, DH), jnp.float32),
        jax.ShapeDtypeStruct((NPAD, DH), jnp.float32),
    )

    @functools.partial(
        pl.kernel, mesh=mesh, out_type=out_type,
        scratch_types=[
            pltpu.VMEM_SHARED((NPAD, DH), jnp.float32),
            pltpu.VMEM((C,), jnp.int32),
            pltpu.VMEM((C,), jnp.int32),
            pltpu.VMEM((C,), jnp.int32),
            pltpu.VMEM((C,), jnp.int32),
            pltpu.VMEM((C, DH), jnp.float32),
            pltpu.VMEM((C, DH), jnp.float32),
            pltpu.SemaphoreType.DMA,
            pltpu.SemaphoreType.DMA,
            pltpu.SemaphoreType.DMA,
            pltpu.SemaphoreType.DMA,
            pltpu.SemaphoreType.DMA,
            pltpu.SemaphoreType.DMA,
            pltpu.SemaphoreType.DMA,
            pltpu.SemaphoreType.DMA,
        ],
    )
    def k(featL_hbm, featR_hbm, src_hbm, dst_hbm, zacc_hbm, ones_hbm,
          sumL_hbm, sumR_hbm, degA_hbm, degB_hbm,
          acc_sh, src_v0, src_v1, dst_v0, dst_v1, rows_v0, rows_v1,
          isem0, isem1, dsem0, dsem1, gsem0, gsem1, ssem0, ssem1):
        cid = lax.axis_index("c")
        sid = lax.axis_index("s")
        wid = cid * NS + sid
        r0 = sid * RPW
        nchunk = RPW // C
        sv = (src_v0, src_v1)
        dv = (dst_v0, dst_v1)
        rv = (rows_v0, rows_v1)
        isem = (isem0, isem1)
        dsem = (dsem0, dsem1)
        gsem = (gsem0, gsem1)
        ssem = (ssem0, ssem1)
        rows_v = rows_v0
        ones_v = rows_v1

        def zero_acc():
            pltpu.sync_copy(zacc_hbm, rows_v)

            @pl.loop(0, nchunk)
            def _(j):
                pltpu.sync_copy(rows_v, acc_sh.at[pl.ds(r0 + j * C, C)])

        def write_acc(out_hbm):
            @pl.loop(0, nchunk)
            def _(j):
                rr = r0 + j * C
                pltpu.sync_copy(acc_sh.at[pl.ds(rr, C)], rows_v)
                pltpu.sync_copy(rows_v, out_hbm.at[pl.ds(rr, C)])

        zero_acc()
        plsc.subcore_barrier()

        def run(feat_hbm):
            npair = (NITER - 1) // 2

            def isrc_start(i, b):
                pltpu.async_copy(
                    src_hbm.at[pl.ds(sid * EPS + i * C, C)], sv[b], isem[b])

            def isrc_wait(b):
                pltpu.make_async_copy(
                    src_hbm.at[pl.ds(0, C)], sv[b], isem[b]).wait()

            def dst_start(i, b):
                pltpu.async_copy(
                    dst_hbm.at[pl.ds(sid * EPS + i * C, C)], dv[b], dsem[b])

            def dst_wait(b):
                pltpu.make_async_copy(
                    dst_hbm.at[pl.ds(0, C)], dv[b], dsem[b]).wait()

            def gather_start(b):
                pltpu.async_copy(feat_hbm.at[sv[b]], rv[b], gsem[b])

            def gather_wait(b):
                pltpu.make_async_copy(feat_hbm.at[sv[b]], rv[b],
                                      gsem[b]).wait()

            def scat_start(b):
                pltpu.async_copy(rv[b], acc_sh.at[dv[b]], ssem[b], add=True)

            def scat_wait(b):
                pltpu.make_async_copy(rv[b], acc_sh.at[dv[b]],
                                      ssem[b]).wait()

            isrc_start(0, 0)
            dst_start(0, 0)
            isrc_wait(0)
            gather_start(0)
            isrc_start(1, 1)
            dst_start(1, 1)
            gather_wait(0)
            dst_wait(0)
            scat_start(0)
            isrc_wait(1)
            gather_start(1)
            isrc_start(2, 0)

            @pl.loop(0, npair)
            def _(p):
                i = 1 + 2 * p
                gather_wait(1)
                dst_wait(1)
                scat_start(1)
                scat_wait(0)
                dst_start(i + 1, 0)
                isrc_wait(0)
                gather_start(0)

                @pl.when(p < npair - 1)
                def _():
                    isrc_start(i + 2, 1)

                gather_wait(0)
                dst_wait(0)
                scat_start(0)
                scat_wait(1)

                @pl.when(p < npair - 1)
                def _():
                    dst_start(i + 2, 1)
                    isrc_wait(1)
                    gather_start(1)
                    isrc_start(i + 3, 0)

            scat_wait(0)

        @pl.when(cid == 0)
        def _():
            run(featL_hbm)

        @pl.when(cid == 1)
        def _():
            run(featR_hbm)

        plsc.subcore_barrier()

        @pl.when(cid == 0)
        def _():
            write_acc(sumL_hbm)

        @pl.when(cid == 1)
        def _():
            write_acc(sumR_hbm)

        plsc.subcore_barrier()

        zero_acc()
        pltpu.sync_copy(ones_hbm, ones_v)
        plsc.subcore_barrier()

        def d2_start(i, b):
            pltpu.async_copy(
                dst_hbm.at[pl.ds((wid + 32 * i) * C, C)], dv[b], dsem[b])

        def d2_wait(b):
            pltpu.make_async_copy(dst_hbm.at[pl.ds(0, C)], dv[b],
                                  dsem[b]).wait()

        def s2_start(b):
            pltpu.async_copy(ones_v, acc_sh.at[dv[b]], ssem[b], add=True)

        def s2_wait(b):
            pltpu.make_async_copy(ones_v, acc_sh.at[dv[b]], ssem[b]).wait()

        def chunk_exists(i):
            return (wid + 32 * i) < NCHUNKS

        d2_start(0, 0)
        d2_start(1, 1)
        d2_wait(0)
        s2_start(0)

        @pl.loop(0, (P2ITERS - 1) // 2)
        def _(p):
            i = 1 + 2 * p
            s2_wait(0)

            @pl.when(chunk_exists(i + 1))
            def _():
                d2_start(i + 1, 0)

            d2_wait(1)
            s2_start(1)
            s2_wait(1)

            @pl.when(chunk_exists(i + 2))
            def _():
                d2_start(i + 2, 1)

            @pl.when(chunk_exists(i + 1))
            def _():
                d2_wait(0)
                s2_start(0)

        @pl.when(chunk_exists(P2ITERS - 1))
        def _():
            s2_wait(0)

        plsc.subcore_barrier()

        @pl.when(cid == 0)
        def _():
            write_acc(degA_hbm)

        @pl.when(cid == 1)
        def _():
            write_acc(degB_hbm)

    return k(featL, featR, src, dst, zacc, ones128)


_DN = (((1,), (1,)), ((), ()))


def _tc_hself(feat, W_self, b_self):
    B = 400

    def body(feat_ref, ws_ref, b_ref, o_ref):
        o_ref[...] = lax.dot_general(
            feat_ref[...], ws_ref[...], _DN,
            preferred_element_type=jnp.float32) + b_ref[...]

    return pl.pallas_call(
        body,
        grid=(N // B,),
        in_specs=[
            pl.BlockSpec((B, D), lambda i: (i, 0)),
            pl.BlockSpec((D, D), lambda i: (0, 0)),
            pl.BlockSpec((1, D), lambda i: (0, 0)),
        ],
        out_specs=pl.BlockSpec((B, D), lambda i: (i, 0)),
        out_shape=jax.ShapeDtypeStruct((N, D), jnp.float32),
    )(feat, W_self, b_self.reshape(1, D))


def _tc_combine(h_self, sumL, sumR, degA, degB, W_neigh):
    B = 400

    def body(hs_ref, sL_ref, sR_ref, dA_ref, dB_ref, wn_ref, o_ref):
        ns = jnp.concatenate([sL_ref[...], sR_ref[...]], axis=1)
        deg = jnp.maximum(dA_ref[...][:, 0:1] + dB_ref[...][:, 0:1], 1.0)
        h_neigh = ns / deg
        o_ref[...] = hs_ref[...] + lax.dot_general(
            h_neigh, wn_ref[...], _DN, preferred_element_type=jnp.float32)

    return pl.pallas_call(
        body,
        grid=(N // B,),
        in_specs=[
            pl.BlockSpec((B, D), lambda i: (i, 0)),
            pl.BlockSpec((B, DH), lambda i: (i, 0)),
            pl.BlockSpec((B, DH), lambda i: (i, 0)),
            pl.BlockSpec((B, DH), lambda i: (i, 0)),
            pl.BlockSpec((B, DH), lambda i: (i, 0)),
            pl.BlockSpec((D, D), lambda i: (0, 0)),
        ],
        out_specs=pl.BlockSpec((B, D), lambda i: (i, 0)),
        out_shape=jax.ShapeDtypeStruct((N, D), jnp.float32),
    )(h_self, sumL, sumR, degA, degB, W_neigh)


def kernel(feat, edge_index, W_neigh, W_self, b_self):
    src = edge_index[0]
    dst = edge_index[1]
    featL = feat[:, :DH]
    featR = feat[:, DH:]
    zacc = jnp.zeros((C, DH), jnp.float32)
    ones128 = jnp.ones((C, DH), jnp.float32)
    sumL, sumR, degA, degB = _sc_segment_sum(featL, featR, src, dst,
                                             zacc, ones128)
    h_self = _tc_hself(feat, W_self, b_self)
    return _tc_combine(h_self, sumL, sumR, degA, degB, W_neigh)
